# baseline (speedup 1.0000x reference)
import functools

import jax
import jax.numpy as jnp
from jax import lax
from jax.experimental import pallas as pl
from jax.experimental.pallas import tpu as pltpu
from jax.experimental.pallas import tpu_sc as plsc

N = 10000
NC = 2
NS = 16
BM = 1000
NI = N // BM
RPT = 624
RPT_LAST = N - (NS - 1) * RPT
CFG_BY_FH = {16: (128, 8), 32: (128, 8), 64: (64, 12), 96: (64, 8),
             128: (64, 6)}


@functools.cache
def _spmm(fh: int, ch: int, nchp: int, k: int):
    nround = nchp // k
    mesh = plsc.VectorSubcoreMesh(
        core_axis_name="c", subcore_axis_name="s", num_cores=NC,
        num_subcores=NS)

    @functools.partial(
        pl.kernel,
        out_type=jax.ShapeDtypeStruct((NC * N, fh), jnp.float32),
        mesh=mesh,
        scratch_types=[
            pltpu.VMEM((2 * k, ch), jnp.int32),
            pltpu.VMEM((2 * k, ch), jnp.int32),
            pltpu.VMEM((k, ch, fh), jnp.float32),
            pltpu.VMEM_SHARED((N + 8, fh), jnp.float32),
            pltpu.SemaphoreType.DMA((k,)),
            pltpu.SemaphoreType.DMA((k,)),
            pltpu.SemaphoreType.DMA,
        ],
        compiler_params=pltpu.CompilerParams(use_tc_tiling_on_sc=False),
    )
    def spmm(h2, src2, dst2, zeros, out,
             src_v, dst_v, rows_v, agg_s, gsem, ssem, isem):
        c = lax.axis_index("c")
        s = lax.axis_index("s")
        base_s = (c * NS + s) * nchp
        base_d = s * nchp
        @pl.when(s < NS - 1)
        def _():
            pltpu.sync_copy(zeros.at[pl.ds(0, RPT)],
                            agg_s.at[pl.ds(s * RPT, RPT)])

        @pl.when(s == NS - 1)
        def _():
            pltpu.sync_copy(zeros, agg_s.at[pl.ds((NS - 1) * RPT, RPT_LAST)])

        plsc.subcore_barrier()

        pltpu.sync_copy(src2.at[pl.ds(base_s, k)], src_v.at[pl.ds(0, k)])
        pltpu.sync_copy(dst2.at[pl.ds(base_d, k)], dst_v.at[pl.ds(0, k)])
        for b in range(k):
            pltpu.async_copy(h2.at[src_v.at[b]], rows_v.at[b], gsem.at[b])

        def round_(r, carry):
            p = lax.rem(r, 2)
            po = p * k
            pno = (1 - p) * k
            nxt = r + 1 < nround

            @pl.when(nxt)
            def _():
                pltpu.async_copy(src2.at[pl.ds(base_s + (r + 1) * k, k)],
                                 src_v.at[pl.ds(pno, k)], isem)
                pltpu.async_copy(dst2.at[pl.ds(base_d + (r + 1) * k, k)],
                                 dst_v.at[pl.ds(pno, k)], isem)

            for b in range(k):
                pltpu.make_async_copy(h2.at[src_v.at[po + b]], rows_v.at[b],
                                      gsem.at[b]).wait()
                pltpu.async_copy(rows_v.at[b], agg_s.at[dst_v.at[po + b]],
                                 ssem.at[b], add=True)

            @pl.when(nxt)
            def _():
                pltpu.make_async_copy(src2.at[pl.ds(base_s + (r + 1) * k, k)],
                                      src_v.at[pl.ds(pno, k)], isem).wait()
                pltpu.make_async_copy(dst2.at[pl.ds(base_d + (r + 1) * k, k)],
                                      dst_v.at[pl.ds(pno, k)], isem).wait()

            for b in range(k):
                pltpu.make_async_copy(rows_v.at[b],
                                      agg_s.at[dst_v.at[po + b]],
                                      ssem.at[b]).wait()

                @pl.when(nxt)
                def _():
                    pltpu.async_copy(h2.at[src_v.at[pno + b]], rows_v.at[b],
                                     gsem.at[b])

            return carry

        lax.fori_loop(0, nround, round_, 0)
        plsc.subcore_barrier()

        @pl.when(s < NS - 1)
        def _():
            pltpu.sync_copy(agg_s.at[pl.ds(s * RPT, RPT)],
                            out.at[pl.ds(c * N + s * RPT, RPT)])

        @pl.when(s == NS - 1)
        def _():
            pltpu.sync_copy(
                agg_s.at[pl.ds((NS - 1) * RPT, RPT_LAST)],
                out.at[pl.ds(c * N + (NS - 1) * RPT, RPT_LAST)])

    return spmm


def _stack_cols(w):
    f = w.shape[1]
    return jnp.stack([w[:, :f // 2], w[:, f // 2:]])


def _mm(ins, dinv, f, row=None):
    fh = f // 2
    in_specs, args, plan = [], [], []
    for arr, w, is_split in ins:
        if is_split:
            fa = arr.shape[1]
            in_specs += [
                pl.BlockSpec((BM, fa), lambda c, i: (i, 0)),
                pl.BlockSpec((BM, fa), lambda c, i: (NI + i, 0)),
                pl.BlockSpec((1, fa, fh), lambda c, i: (c, 0, 0)),
                pl.BlockSpec((1, fa, fh), lambda c, i: (c, 0, 0)),
            ]
            args += [arr, arr, _stack_cols(w[:fa]), _stack_cols(w[fa:])]
            plan.append(("split",))
        else:
            fa = arr.shape[1]
            in_specs += [
                pl.BlockSpec((BM, fa), lambda c, i: (i, 0)),
                pl.BlockSpec((1, fa, fh), lambda c, i: (c, 0, 0)),
            ]
            args += [arr, _stack_cols(w)]
            plan.append(("merged",))
    if row is not None:
        mv, wr = row
        fm = mv.shape[1]
        in_specs += [
            pl.BlockSpec((1, fm), lambda c, i: (0, 0)),
            pl.BlockSpec((1, fm, fh), lambda c, i: (c, 0, 0)),
        ]
        args += [mv, _stack_cols(wr)]
        plan.append(("row",))
    in_specs.append(pl.BlockSpec((BM, 1), lambda c, i: (i, 0)))
    args.append(dinv)

    def body(*refs):
        it = iter(refs)
        acc = None
        for kind, in plan:
            if kind == "split":
                a0, a1, w0, w1 = next(it), next(it), next(it), next(it)
                t = jnp.dot(a0[...], w0[0],
                            preferred_element_type=jnp.float32)
                t += jnp.dot(a1[...], w1[0],
                             preferred_element_type=jnp.float32)
            elif kind == "merged":
                a_r, w_r = next(it), next(it)
                t = jnp.dot(a_r[...], w_r[0],
                            preferred_element_type=jnp.float32)
            else:
                mv_r, wr_r = next(it), next(it)
                t = jnp.dot(mv_r[...], wr_r[0],
                            preferred_element_type=jnp.float32)
            acc = t if acc is None else acc + t
        d_r = next(it)
        o_r = next(it)
        o_r[...] = acc * d_r[...]

    return pl.pallas_call(
        body,
        grid=(NC, NI),
        in_specs=in_specs,
        out_specs=pl.BlockSpec((BM, fh), lambda c, i: (c * NI + i, 0)),
        out_shape=jax.ShapeDtypeStruct((NC * N, fh), jnp.float32),
    )(*args)


def _epi(agg2, h2, dinv, bias, fill, act, mul=None, add=None, colmax=None):
    fh = agg2.shape[1]

    def body(*refs):
        it = iter(refs)
        agg_r = next(it)
        h_r = next(it)
        d_r = next(it)
        b_r = next(it)
        mul_r = next(it) if mul is not None else None
        add_r = next(it) if add is not None else None
        o_r = next(it)
        cm_r = next(it) if colmax else None
        t = (agg_r[...] + fill * h_r[...]) * d_r[...] + b_r[0]
        if act == "tanh":
            t = jnp.tanh(t)
        elif act == "sig_tanh":
            t = jax.nn.sigmoid(jnp.tanh(t))
        elif act == "tanh_tanh":
            t = jnp.tanh(jnp.tanh(t))
        if colmax:
            c = pl.program_id(0)
            i = pl.program_id(1)
            on = (c == 0) if colmax == "half" else (c >= 0)
            cm = cm_r if colmax == "half" else cm_r.at[0]

            @pl.when(on & (i == 0))
            def _():
                cm[...] = jnp.max(t, axis=0, keepdims=True)

            @pl.when(on & (i > 0))
            def _():
                cm[...] = jnp.maximum(cm[...],
                                      jnp.max(t, axis=0, keepdims=True))
        if mul_r is not None:
            t = t * mul_r[...]
        if add_r is not None:
            t = t + add_r[...]
        o_r[...] = t

    split_spec = pl.BlockSpec((BM, fh), lambda c, i: (c * NI + i, 0))
    in_specs = [
        split_spec,
        split_spec,
        pl.BlockSpec((BM, 1), lambda c, i: (i, 0)),
        pl.BlockSpec((1, 1, fh), lambda c, i: (c, 0, 0)),
    ]
    args = [agg2, h2, dinv, _stack_cols(bias).reshape(2, 1, fh)]
    if mul is not None:
        in_specs.append(split_spec)
        args.append(mul)
    if add is not None:
        in_specs.append(split_spec)
        args.append(add)
    out_specs = [split_spec]
    out_shape = [jax.ShapeDtypeStruct((NC * N, fh), jnp.float32)]
    if colmax == "half":
        out_specs.append(pl.BlockSpec((1, fh), lambda c, i: (0, 0)))
        out_shape.append(jax.ShapeDtypeStruct((1, fh), jnp.float32))
    elif colmax == "full":
        out_specs.append(pl.BlockSpec((1, 1, fh), lambda c, i: (c, 0, 0)))
        out_shape.append(jax.ShapeDtypeStruct((NC, 1, fh), jnp.float32))
    res = pl.pallas_call(
        body,
        grid=(NC, NI),
        in_specs=in_specs,
        out_specs=out_specs,
        out_shape=out_shape,
    )(*args)
    if colmax == "full":
        return res[0], res[1].reshape(1, 2 * fh)
    return res if colmax else res[0]


def _dinv_pair(indeg):
    def body(ind_r, d1_r, d2_r):
        ind = ind_r[...]
        d1_r[...] = lax.rsqrt(ind + 1.0)
        d2_r[...] = lax.rsqrt(ind + 2.0)

    spec = pl.BlockSpec((BM, 1), lambda i: (i, 0))
    return pl.pallas_call(
        body,
        grid=(NI,),
        in_specs=[spec],
        out_specs=[spec, spec],
        out_shape=[jax.ShapeDtypeStruct((N, 1), jnp.float32)] * 2,
    )(indeg)


def _merge(h2):
    fh = h2.shape[1]

    def body(a_r, b_r, o_r):
        o_r[...] = jnp.concatenate([a_r[...], b_r[...]], axis=1)

    return pl.pallas_call(
        body,
        grid=(NI,),
        in_specs=[
            pl.BlockSpec((BM, fh), lambda i: (i, 0)),
            pl.BlockSpec((BM, fh), lambda i: (NI + i, 0)),
        ],
        out_specs=pl.BlockSpec((BM, 2 * fh), lambda i: (i, 0)),
        out_shape=jax.ShapeDtypeStruct((N, 2 * fh), jnp.float32),
    )(h2, h2)


def _run_spmm(graph, h2):
    fh = h2.shape[1]
    ch, k = CFG_BY_FH[fh]
    src2, dst2, nchp = graph[(ch, k)]
    zeros = jnp.zeros((RPT_LAST, fh), jnp.float32)
    return _spmm(fh, ch, nchp, k)(h2, src2, dst2, zeros)


def _gcn_step(graph, ins, bias, dinv, fill, act, row=None, mul=None,
              add=None, colmax=None):
    f = ins[0][1].shape[1]
    h2 = _mm(ins, dinv, f, row=row)
    agg2 = _run_spmm(graph, h2)
    return _epi(agg2, h2, dinv, bias.reshape(1, f), fill, act,
                mul=mul, add=add, colmax=colmax)


def _pad_rows(w, top, total):
    out = jnp.zeros((total, w.shape[1]), w.dtype)
    return out.at[top:top + w.shape[0]].set(w)


def _inception(graph, p, x, dinv, act="tanh", mul=None, add=None):
    d2 = p["conv1"]["W"].shape[1]
    d4 = 2 * d2
    w13 = jnp.concatenate([p["conv1"]["W"], p["conv3"]["W"]], axis=1)
    b13 = jnp.concatenate([p["conv1"]["b"], p["conv3"]["b"]])
    t13 = _gcn_step(graph, [(x, w13, True)], b13, dinv, 1.0, "tanh")
    w7 = p["conv7"]["W"]
    if 2 * d4 <= 384:
        wbd = jnp.concatenate([
            _pad_rows(p["conv2"]["W"], 0, d4),
            _pad_rows(p["conv4"]["W"], d2, d4)], axis=1)
        b24 = jnp.concatenate([p["conv2"]["b"], p["conv4"]["b"]])
        t24, mv = _gcn_step(graph, [(t13, wbd, True)], b24, dinv, 1.0,
                            "tanh", colmax="half")
        i2src, wi2 = t24, _pad_rows(w7[d4:2 * d4], d4, 2 * d4)
    else:
        t2, mv = _gcn_step(graph,
                           [(t13, _pad_rows(p["conv2"]["W"], 0, d4), True)],
                           p["conv2"]["b"], dinv, 1.0, "tanh", colmax="full")
        t4 = _gcn_step(graph,
                       [(t13, _pad_rows(p["conv4"]["W"], d2, d4), True)],
                       p["conv4"]["b"], dinv, 1.0, "tanh")
        i2src, wi2 = t4, w7[d4:2 * d4]
    return _gcn_step(graph, [(i2src, wi2, True), (x, w7[2 * d4:], True)],
                     p["conv7"]["b"], dinv, 1.0, act,
                     row=(mv, w7[:d4]), mul=mul, add=add)


def _lstm(graph, p, x, x_split, dinv):
    xl = _gcn_step(graph, [(x, p["conv1"]["W"], x_split)],
                   p["conv1"]["b"], dinv, 1.0, "none")
    f1 = _inception(graph, p["inc1"], xl, dinv)
    f = _inception(graph, p["inc2"], f1, dinv, mul=xl)
    i1s = _inception(graph, p["inc3"], xl, dinv)
    i1 = _inception(graph, p["inc4"], i1s, dinv, act="sig_tanh")
    i2s = _inception(graph, p["inc5"], xl, dinv)
    i = _inception(graph, p["inc6"], i2s, dinv, act="tanh_tanh",
                   mul=i1, add=f)
    return _gcn_step(graph, [(i, p["conv2"]["W"], True)],
                     p["conv2"]["b"], dinv, 1.0, "tanh")


def kernel(x, adj_t, batch, params):
    del batch
    src = adj_t[0].astype(jnp.int32)
    dst = adj_t[1].astype(jnp.int32)
    e = src.shape[0]
    assert e % NS == 0 and N % NS == 0 and N % BM == 0
    ept = e // NS
    graph = {}
    for ch, k in sorted(set(CFG_BY_FH.values())):
        nchp = -(-ept // ch)
        nchp = -(-nchp // k) * k
        pad = nchp * ch - ept
        pad_src = jnp.broadcast_to(jnp.arange(pad, dtype=jnp.int32) % N,
                                   (NS, pad))
        s_t = jnp.concatenate([src.reshape(NS, ept), pad_src], axis=1)
        d_t = jnp.pad(dst.reshape(NS, ept), ((0, 0), (0, pad)),
                      constant_values=N)
        src2 = jnp.concatenate([s_t, s_t + N]).reshape(2 * NS * nchp, ch)
        dst2 = d_t.reshape(NS * nchp, ch)
        graph[(ch, k)] = (src2, dst2, nchp)

    ones2 = jnp.ones((NC * N, 16), jnp.float32)
    indeg2 = _run_spmm(graph, ones2)
    dinv1, dinv2 = _dinv_pair(indeg2[:N, :1])

    h = _lstm(graph, params["lstm1"], x, False, dinv1)
    h = _lstm(graph, params["lstm2"], h, True, dinv1)
    wt = params["conv1"]["W"]
    fx = x.shape[1]
    h = _gcn_step(graph, [(x, wt[:fx], False), (h, wt[fx:], True)],
                  params["conv1"]["b"], dinv2, 2.0, "tanh")
    h = _gcn_step(graph, [(h, params["conv2"]["W"], True)],
                  params["conv2"]["b"], dinv2, 2.0, "tanh")
    h = _gcn_step(graph, [(h, params["conv3"]["W"], True)],
                  params["conv3"]["b"], dinv2, 2.0, "tanh")
    h = _gcn_step(graph, [(h, params["conv4"]["W"], True)],
                  params["conv4"]["b"], dinv2, 2.0, "tanh")
    out2 = _gcn_step(graph, [(h, params["convOut"]["W"], True)],
                     params["convOut"]["b"], dinv2, 2.0, "none")
    return _merge(out2)


# speedup vs baseline: 1.0099x; 1.0099x over previous
import functools

import jax
import jax.numpy as jnp
from jax import lax
from jax.experimental import pallas as pl
from jax.experimental.pallas import tpu as pltpu
from jax.experimental.pallas import tpu_sc as plsc

N = 10000
NC = 2
NS = 16
BM = 1000
NI = N // BM
RPT = 624
RPT_LAST = N - (NS - 1) * RPT
CFG_BY_FH = {16: (128, 8), 32: (128, 8), 64: (128, 8), 96: (64, 8),
             128: (56, 6)}


@functools.cache
def _spmm(fh: int, ch: int, nchp: int, k: int):
    nround = nchp // k
    mesh = plsc.VectorSubcoreMesh(
        core_axis_name="c", subcore_axis_name="s", num_cores=NC,
        num_subcores=NS)

    @functools.partial(
        pl.kernel,
        out_type=jax.ShapeDtypeStruct((NC * N, fh), jnp.float32),
        mesh=mesh,
        scratch_types=[
            pltpu.VMEM((2 * k, ch), jnp.int32),
            pltpu.VMEM((2 * k, ch), jnp.int32),
            pltpu.VMEM((k, ch, fh), jnp.float32),
            pltpu.VMEM_SHARED((N + 8, fh), jnp.float32),
            pltpu.SemaphoreType.DMA((k,)),
            pltpu.SemaphoreType.DMA((k,)),
            pltpu.SemaphoreType.DMA,
        ],
        compiler_params=pltpu.CompilerParams(use_tc_tiling_on_sc=False),
    )
    def spmm(h2, src2, dst2, zeros, out,
             src_v, dst_v, rows_v, agg_s, gsem, ssem, isem):
        c = lax.axis_index("c")
        s = lax.axis_index("s")
        base_s = (c * NS + s) * nchp
        base_d = s * nchp
        @pl.when(s < NS - 1)
        def _():
            pltpu.sync_copy(zeros.at[pl.ds(0, RPT)],
                            agg_s.at[pl.ds(s * RPT, RPT)])

        @pl.when(s == NS - 1)
        def _():
            pltpu.sync_copy(zeros, agg_s.at[pl.ds((NS - 1) * RPT, RPT_LAST)])

        plsc.subcore_barrier()

        pltpu.sync_copy(src2.at[pl.ds(base_s, k)], src_v.at[pl.ds(0, k)])
        pltpu.sync_copy(dst2.at[pl.ds(base_d, k)], dst_v.at[pl.ds(0, k)])
        for b in range(k):
            pltpu.async_copy(h2.at[src_v.at[b]], rows_v.at[b], gsem.at[b])

        def round_(r, carry):
            p = lax.rem(r, 2)
            po = p * k
            pno = (1 - p) * k
            nxt = r + 1 < nround

            @pl.when(nxt)
            def _():
                pltpu.async_copy(src2.at[pl.ds(base_s + (r + 1) * k, k)],
                                 src_v.at[pl.ds(pno, k)], isem)
                pltpu.async_copy(dst2.at[pl.ds(base_d + (r + 1) * k, k)],
                                 dst_v.at[pl.ds(pno, k)], isem)

            for b in range(k):
                pltpu.make_async_copy(h2.at[src_v.at[po + b]], rows_v.at[b],
                                      gsem.at[b]).wait()
                pltpu.async_copy(rows_v.at[b], agg_s.at[dst_v.at[po + b]],
                                 ssem.at[b], add=True)

            @pl.when(nxt)
            def _():
                pltpu.make_async_copy(src2.at[pl.ds(base_s + (r + 1) * k, k)],
                                      src_v.at[pl.ds(pno, k)], isem).wait()
                pltpu.make_async_copy(dst2.at[pl.ds(base_d + (r + 1) * k, k)],
                                      dst_v.at[pl.ds(pno, k)], isem).wait()

            for b in range(k):
                pltpu.make_async_copy(rows_v.at[b],
                                      agg_s.at[dst_v.at[po + b]],
                                      ssem.at[b]).wait()

                @pl.when(nxt)
                def _():
                    pltpu.async_copy(h2.at[src_v.at[pno + b]], rows_v.at[b],
                                     gsem.at[b])

            return carry

        lax.fori_loop(0, nround, round_, 0)
        plsc.subcore_barrier()

        @pl.when(s < NS - 1)
        def _():
            pltpu.sync_copy(agg_s.at[pl.ds(s * RPT, RPT)],
                            out.at[pl.ds(c * N + s * RPT, RPT)])

        @pl.when(s == NS - 1)
        def _():
            pltpu.sync_copy(
                agg_s.at[pl.ds((NS - 1) * RPT, RPT_LAST)],
                out.at[pl.ds(c * N + (NS - 1) * RPT, RPT_LAST)])

    return spmm


def _stack_cols(w):
    f = w.shape[1]
    return jnp.stack([w[:, :f // 2], w[:, f // 2:]])


def _mm(ins, dinv, f, row=None):
    fh = f // 2
    in_specs, args, plan = [], [], []
    for arr, w, is_split in ins:
        if is_split:
            fa = arr.shape[1]
            in_specs += [
                pl.BlockSpec((BM, fa), lambda c, i: (i, 0)),
                pl.BlockSpec((BM, fa), lambda c, i: (NI + i, 0)),
                pl.BlockSpec((1, fa, fh), lambda c, i: (c, 0, 0)),
                pl.BlockSpec((1, fa, fh), lambda c, i: (c, 0, 0)),
            ]
            args += [arr, arr, _stack_cols(w[:fa]), _stack_cols(w[fa:])]
            plan.append(("split",))
        else:
            fa = arr.shape[1]
            in_specs += [
                pl.BlockSpec((BM, fa), lambda c, i: (i, 0)),
                pl.BlockSpec((1, fa, fh), lambda c, i: (c, 0, 0)),
            ]
            args += [arr, _stack_cols(w)]
            plan.append(("merged",))
    if row is not None:
        mv, wr = row
        fm = mv.shape[1]
        in_specs += [
            pl.BlockSpec((1, fm), lambda c, i: (0, 0)),
            pl.BlockSpec((1, fm, fh), lambda c, i: (c, 0, 0)),
        ]
        args += [mv, _stack_cols(wr)]
        plan.append(("row",))
    in_specs.append(pl.BlockSpec((BM, 1), lambda c, i: (i, 0)))
    args.append(dinv)

    def body(*refs):
        it = iter(refs)
        acc = None
        for kind, in plan:
            if kind == "split":
                a0, a1, w0, w1 = next(it), next(it), next(it), next(it)
                t = jnp.dot(a0[...], w0[0],
                            preferred_element_type=jnp.float32)
                t += jnp.dot(a1[...], w1[0],
                             preferred_element_type=jnp.float32)
            elif kind == "merged":
                a_r, w_r = next(it), next(it)
                t = jnp.dot(a_r[...], w_r[0],
                            preferred_element_type=jnp.float32)
            else:
                mv_r, wr_r = next(it), next(it)
                t = jnp.dot(mv_r[...], wr_r[0],
                            preferred_element_type=jnp.float32)
            acc = t if acc is None else acc + t
        d_r = next(it)
        o_r = next(it)
        o_r[...] = acc * d_r[...]

    return pl.pallas_call(
        body,
        grid=(NC, NI),
        in_specs=in_specs,
        out_specs=pl.BlockSpec((BM, fh), lambda c, i: (c * NI + i, 0)),
        out_shape=jax.ShapeDtypeStruct((NC * N, fh), jnp.float32),
    )(*args)


def _epi(agg2, h2, dinv, bias, fill, act, mul=None, add=None, colmax=None):
    fh = agg2.shape[1]

    def body(*refs):
        it = iter(refs)
        agg_r = next(it)
        h_r = next(it)
        d_r = next(it)
        b_r = next(it)
        mul_r = next(it) if mul is not None else None
        add_r = next(it) if add is not None else None
        o_r = next(it)
        cm_r = next(it) if colmax else None
        t = (agg_r[...] + fill * h_r[...]) * d_r[...] + b_r[0]
        if act == "tanh":
            t = jnp.tanh(t)
        elif act == "sig_tanh":
            t = jax.nn.sigmoid(jnp.tanh(t))
        elif act == "tanh_tanh":
            t = jnp.tanh(jnp.tanh(t))
        if colmax:
            c = pl.program_id(0)
            i = pl.program_id(1)
            on = (c == 0) if colmax == "half" else (c >= 0)
            cm = cm_r if colmax == "half" else cm_r.at[0]

            @pl.when(on & (i == 0))
            def _():
                cm[...] = jnp.max(t, axis=0, keepdims=True)

            @pl.when(on & (i > 0))
            def _():
                cm[...] = jnp.maximum(cm[...],
                                      jnp.max(t, axis=0, keepdims=True))
        if mul_r is not None:
            t = t * mul_r[...]
        if add_r is not None:
            t = t + add_r[...]
        o_r[...] = t

    split_spec = pl.BlockSpec((BM, fh), lambda c, i: (c * NI + i, 0))
    in_specs = [
        split_spec,
        split_spec,
        pl.BlockSpec((BM, 1), lambda c, i: (i, 0)),
        pl.BlockSpec((1, 1, fh), lambda c, i: (c, 0, 0)),
    ]
    args = [agg2, h2, dinv, _stack_cols(bias).reshape(2, 1, fh)]
    if mul is not None:
        in_specs.append(split_spec)
        args.append(mul)
    if add is not None:
        in_specs.append(split_spec)
        args.append(add)
    out_specs = [split_spec]
    out_shape = [jax.ShapeDtypeStruct((NC * N, fh), jnp.float32)]
    if colmax == "half":
        out_specs.append(pl.BlockSpec((1, fh), lambda c, i: (0, 0)))
        out_shape.append(jax.ShapeDtypeStruct((1, fh), jnp.float32))
    elif colmax == "full":
        out_specs.append(pl.BlockSpec((1, 1, fh), lambda c, i: (c, 0, 0)))
        out_shape.append(jax.ShapeDtypeStruct((NC, 1, fh), jnp.float32))
    res = pl.pallas_call(
        body,
        grid=(NC, NI),
        in_specs=in_specs,
        out_specs=out_specs,
        out_shape=out_shape,
    )(*args)
    if colmax == "full":
        return res[0], res[1].reshape(1, 2 * fh)
    return res if colmax else res[0]


def _dinv_pair(indeg):
    def body(ind_r, d1_r, d2_r):
        ind = ind_r[...]
        d1_r[...] = lax.rsqrt(ind + 1.0)
        d2_r[...] = lax.rsqrt(ind + 2.0)

    spec = pl.BlockSpec((BM, 1), lambda i: (i, 0))
    return pl.pallas_call(
        body,
        grid=(NI,),
        in_specs=[spec],
        out_specs=[spec, spec],
        out_shape=[jax.ShapeDtypeStruct((N, 1), jnp.float32)] * 2,
    )(indeg)


def _merge(h2):
    fh = h2.shape[1]

    def body(a_r, b_r, o_r):
        o_r[...] = jnp.concatenate([a_r[...], b_r[...]], axis=1)

    return pl.pallas_call(
        body,
        grid=(NI,),
        in_specs=[
            pl.BlockSpec((BM, fh), lambda i: (i, 0)),
            pl.BlockSpec((BM, fh), lambda i: (NI + i, 0)),
        ],
        out_specs=pl.BlockSpec((BM, 2 * fh), lambda i: (i, 0)),
        out_shape=jax.ShapeDtypeStruct((N, 2 * fh), jnp.float32),
    )(h2, h2)


def _run_spmm(graph, h2):
    fh = h2.shape[1]
    ch, k = CFG_BY_FH[fh]
    src2, dst2, nchp = graph[(ch, k)]
    zeros = jnp.zeros((RPT_LAST, fh), jnp.float32)
    return _spmm(fh, ch, nchp, k)(h2, src2, dst2, zeros)


def _gcn_step(graph, ins, bias, dinv, fill, act, row=None, mul=None,
              add=None, colmax=None):
    f = ins[0][1].shape[1]
    h2 = _mm(ins, dinv, f, row=row)
    agg2 = _run_spmm(graph, h2)
    return _epi(agg2, h2, dinv, bias.reshape(1, f), fill, act,
                mul=mul, add=add, colmax=colmax)


def _pad_rows(w, top, total):
    out = jnp.zeros((total, w.shape[1]), w.dtype)
    return out.at[top:top + w.shape[0]].set(w)


def _inception(graph, p, x, dinv, act="tanh", mul=None, add=None):
    d2 = p["conv1"]["W"].shape[1]
    d4 = 2 * d2
    w13 = jnp.concatenate([p["conv1"]["W"], p["conv3"]["W"]], axis=1)
    b13 = jnp.concatenate([p["conv1"]["b"], p["conv3"]["b"]])
    t13 = _gcn_step(graph, [(x, w13, True)], b13, dinv, 1.0, "tanh")
    w7 = p["conv7"]["W"]
    if 2 * d4 <= 384:
        wbd = jnp.concatenate([
            _pad_rows(p["conv2"]["W"], 0, d4),
            _pad_rows(p["conv4"]["W"], d2, d4)], axis=1)
        b24 = jnp.concatenate([p["conv2"]["b"], p["conv4"]["b"]])
        t24, mv = _gcn_step(graph, [(t13, wbd, True)], b24, dinv, 1.0,
                            "tanh", colmax="half")
        i2src, wi2 = t24, _pad_rows(w7[d4:2 * d4], d4, 2 * d4)
    else:
        t2, mv = _gcn_step(graph,
                           [(t13, _pad_rows(p["conv2"]["W"], 0, d4), True)],
                           p["conv2"]["b"], dinv, 1.0, "tanh", colmax="full")
        t4 = _gcn_step(graph,
                       [(t13, _pad_rows(p["conv4"]["W"], d2, d4), True)],
                       p["conv4"]["b"], dinv, 1.0, "tanh")
        i2src, wi2 = t4, w7[d4:2 * d4]
    return _gcn_step(graph, [(i2src, wi2, True), (x, w7[2 * d4:], True)],
                     p["conv7"]["b"], dinv, 1.0, act,
                     row=(mv, w7[:d4]), mul=mul, add=add)


def _lstm(graph, p, x, x_split, dinv):
    xl = _gcn_step(graph, [(x, p["conv1"]["W"], x_split)],
                   p["conv1"]["b"], dinv, 1.0, "none")
    f1 = _inception(graph, p["inc1"], xl, dinv)
    f = _inception(graph, p["inc2"], f1, dinv, mul=xl)
    i1s = _inception(graph, p["inc3"], xl, dinv)
    i1 = _inception(graph, p["inc4"], i1s, dinv, act="sig_tanh")
    i2s = _inception(graph, p["inc5"], xl, dinv)
    i = _inception(graph, p["inc6"], i2s, dinv, act="tanh_tanh",
                   mul=i1, add=f)
    return _gcn_step(graph, [(i, p["conv2"]["W"], True)],
                     p["conv2"]["b"], dinv, 1.0, "tanh")


def kernel(x, adj_t, batch, params):
    del batch
    src = adj_t[0].astype(jnp.int32)
    dst = adj_t[1].astype(jnp.int32)
    e = src.shape[0]
    assert e % NS == 0 and N % NS == 0 and N % BM == 0
    ept = e // NS
    graph = {}
    for ch, k in sorted(set(CFG_BY_FH.values())):
        nchp = -(-ept // ch)
        nchp = -(-nchp // k) * k
        pad = nchp * ch - ept
        pad_src = jnp.broadcast_to(jnp.arange(pad, dtype=jnp.int32) % N,
                                   (NS, pad))
        s_t = jnp.concatenate([src.reshape(NS, ept), pad_src], axis=1)
        d_t = jnp.pad(dst.reshape(NS, ept), ((0, 0), (0, pad)),
                      constant_values=N)
        src2 = jnp.concatenate([s_t, s_t + N]).reshape(2 * NS * nchp, ch)
        dst2 = d_t.reshape(NS * nchp, ch)
        graph[(ch, k)] = (src2, dst2, nchp)

    ones2 = jnp.ones((NC * N, 16), jnp.float32)
    indeg2 = _run_spmm(graph, ones2)
    dinv1, dinv2 = _dinv_pair(indeg2[:N, :1])

    h = _lstm(graph, params["lstm1"], x, False, dinv1)
    h = _lstm(graph, params["lstm2"], h, True, dinv1)
    wt = params["conv1"]["W"]
    fx = x.shape[1]
    h = _gcn_step(graph, [(x, wt[:fx], False), (h, wt[fx:], True)],
                  params["conv1"]["b"], dinv2, 2.0, "tanh")
    h = _gcn_step(graph, [(h, params["conv2"]["W"], True)],
                  params["conv2"]["b"], dinv2, 2.0, "tanh")
    h = _gcn_step(graph, [(h, params["conv3"]["W"], True)],
                  params["conv3"]["b"], dinv2, 2.0, "tanh")
    h = _gcn_step(graph, [(h, params["conv4"]["W"], True)],
                  params["conv4"]["b"], dinv2, 2.0, "tanh")
    out2 = _gcn_step(graph, [(h, params["convOut"]["W"], True)],
                     params["convOut"]["b"], dinv2, 2.0, "none")
    return _merge(out2)
